# baseline (speedup 1.0000x reference)
import dataclasses
import functools

import jax
import jax.numpy as jnp
from jax import lax
from jax.experimental import pallas as pl
from jax.experimental.pallas import tpu as pltpu
from jax.experimental.pallas import tpu_sc as plsc

N = 10000
D = 128
E = 320000
NEG = 0.2
EPS = 1e-16

CHUNK = 32
EROWS = E // CHUNK
NCORES = 2
NSUB = 16
HC = D // NCORES
RPS = EROWS // NSUB
NPS = N // NSUB

_f32 = jnp.float32


def _dense_stage(x, wl_ref, bl_ref, wr_ref, br_ref, attl_ref, attr_ref,
                 srca_out, srcb_out, al_out, ar_out, amax_out):
    source = jnp.dot(x, wl_ref[...], preferred_element_type=_f32)
    source = source + bl_ref[...][None, :]
    al = jnp.dot(source, attl_ref[...], preferred_element_type=_f32)
    wre = jnp.dot(wr_ref[...], attr_ref[...], preferred_element_type=_f32)
    bre = jnp.sum(br_ref[...] * attr_ref[...][:, 0])
    ar = jnp.dot(x, wre, preferred_element_type=_f32) + bre
    amax_out[...] = jnp.full((1, D), jnp.max(al), _f32)
    srca_out[...] = source[:, :HC]
    srcb_out[...] = source[:, HC:]
    al_out[...] = al
    ar_out[...] = ar


def _prep0_body(x_ref, wl_ref, bl_ref, wr_ref, br_ref, attl_ref, attr_ref,
                srca_out, srcb_out, al_out, ar_out, amax_out):
    _dense_stage(x_ref[...], wl_ref, bl_ref, wr_ref, br_ref, attl_ref,
                 attr_ref, srca_out, srcb_out, al_out, ar_out, amax_out)


def _combine(agg_ref, s_ref):
    recip = 1.0 / (s_ref[...] + EPS)
    h = jnp.concatenate([agg_ref[0], agg_ref[1]], axis=1) * recip
    return jnp.maximum(h, 0.0)


def _prep_body(agg_ref, s_ref, wl_ref, bl_ref, wr_ref, br_ref, attl_ref,
               attr_ref, srca_out, srcb_out, al_out, ar_out, amax_out):
    _dense_stage(_combine(agg_ref, s_ref), wl_ref, bl_ref, wr_ref, br_ref,
                 attl_ref, attr_ref, srca_out, srcb_out, al_out, ar_out,
                 amax_out)


def _final_body(agg_ref, s_ref, wp1_ref, bp1_ref, wp2_ref, bp2_ref, out_ref):
    x = _combine(agg_ref, s_ref)
    h = jnp.dot(x, wp1_ref[...], preferred_element_type=_f32)
    h = h + bp1_ref[...][None, :]
    h = jnp.dot(h, wp2_ref[...], preferred_element_type=_f32)
    out_ref[...] = h + bp2_ref[...][None, :]


_prep_shapes = (
    jax.ShapeDtypeStruct((N, HC), _f32),
    jax.ShapeDtypeStruct((N, HC), _f32),
    jax.ShapeDtypeStruct((N, 1), _f32),
    jax.ShapeDtypeStruct((N, 1), _f32),
    jax.ShapeDtypeStruct((1, D), _f32),
)

_prep0 = pl.pallas_call(_prep0_body, out_shape=_prep_shapes)
_prep = pl.pallas_call(_prep_body, out_shape=_prep_shapes)
_final = pl.pallas_call(_final_body,
                        out_shape=jax.ShapeDtypeStruct((N, D), _f32))


_mesh = plsc.VectorSubcoreMesh(core_axis_name="c", subcore_axis_name="s")

_sc_params = pltpu.CompilerParams()
for _k, _v in (("needs_layout_passes", False), ("use_tc_tiling_on_sc", False)):
    if _k in pltpu.CompilerParams.__dataclass_fields__:
        _sc_params = dataclasses.replace(_sc_params, **{_k: _v})


@functools.partial(
    pl.kernel,
    mesh=_mesh,
    compiler_params=_sc_params,
    out_type=(
        jax.ShapeDtypeStruct((1, N), _f32),
        jax.ShapeDtypeStruct((NCORES, NSUB, NPS, HC), _f32),
    ),
    scratch_types=[
        pltpu.VMEM((RPS, CHUNK), jnp.int32),
        pltpu.VMEM((RPS, CHUNK), jnp.int32),
        pltpu.VMEM((RPS, CHUNK), _f32),
        pltpu.VMEM((N,), _f32),
        pltpu.VMEM((N,), _f32),
        pltpu.VMEM((1, D), _f32),
        pltpu.VMEM((CHUNK, HC), _f32),
        pltpu.VMEM((CHUNK, HC), _f32),
        pltpu.VMEM((CHUNK, HC), _f32),
        pltpu.VMEM((CHUNK, HC), _f32),
        pltpu.VMEM_SHARED((N,), _f32),
        pltpu.VMEM_SHARED((N, HC), _f32),
        pltpu.SemaphoreType.DMA,
        pltpu.SemaphoreType.DMA,
        pltpu.SemaphoreType.DMA,
        pltpu.SemaphoreType.DMA,
        pltpu.SemaphoreType.DMA,
        pltpu.SemaphoreType.DMA,
        pltpu.SemaphoreType.DMA,
        pltpu.SemaphoreType.DMA,
        pltpu.SemaphoreType.DMA,
    ],
)
def _edge_kernel(src_hbm, dst_hbm, al_hbm, ar_hbm, amax_hbm, zn_hbm,
                 srca_hbm, srcb_hbm, s_out, agg_out,
                 src_v, dst_v, t_v, al_v, ar_v, amax_v, rows_a, rows_b,
                 rows_c, rows_d, s_sh, agg_sh, gsem_a, gsem_b, gsem_c, gsem_d,
                 ssem_a, ssem_b, ssem_c, ssem_d, tsem):
    cid = lax.axis_index("c")
    sid = lax.axis_index("s")

    _in_copies = ((src_hbm.at[sid], src_v), (dst_hbm.at[sid], dst_v),
                  (al_hbm, al_v), (ar_hbm, ar_v), (amax_hbm, amax_v))
    for _src, _dst in _in_copies:
        pltpu.async_copy(_src, _dst, tsem)

    @pl.loop(0, CHUNK)
    def _(i):
        for c in range(0, HC, 16):
            rows_a[i, pl.ds(c, 16)] = jnp.zeros((16,), _f32)

    @pl.loop(0, 19)
    def _(k):
        pltpu.async_copy(rows_a, agg_sh.at[pl.ds(sid * NPS + k * CHUNK, CHUNK)],
                         gsem_b)

    pltpu.async_copy(rows_a.at[pl.ds(0, NPS - 19 * CHUNK)],
                     agg_sh.at[pl.ds(sid * NPS + 19 * CHUNK, NPS - 19 * CHUNK)],
                     gsem_a)

    @pl.when(jnp.logical_and(cid == 0, sid == 0))
    def _():
        pltpu.sync_copy(zn_hbm, s_sh)

    for _src, _dst in _in_copies:
        pltpu.make_async_copy(_src, _dst, tsem).wait()

    av = amax_v[0, pl.ds(0, 16)]

    @plsc.parallel_loop(0, RPS, step=1, unroll=2)
    def _(j):
        for k in range(0, CHUNK, 16):
            si = src_v[j, pl.ds(k, 16)]
            di = dst_v[j, pl.ds(k, 16)]
            a = plsc.load_gather(al_v, [si])
            b = plsc.load_gather(ar_v, [di])
            z = a + b
            aw = jnp.where(z >= 0, z, NEG * z)
            z2 = b + av
            g = jnp.where(z2 >= 0, z2, NEG * z2)
            t_v[j, pl.ds(k, 16)] = jnp.exp(aw - g)

    @pl.loop(0, 19)
    def _(k):
        pltpu.make_async_copy(
            rows_a, agg_sh.at[pl.ds(sid * NPS + k * CHUNK, CHUNK)],
            gsem_b).wait()

    pltpu.make_async_copy(rows_a.at[pl.ds(0, NPS - 19 * CHUNK)],
                          agg_sh.at[pl.ds(sid * NPS + 19 * CHUNK,
                                          NPS - 19 * CHUNK)], gsem_a).wait()

    plsc.subcore_barrier()

    @pl.when(cid == 0)
    def _():
        @pl.loop(0, RPS, step=5)
        def _(j):
            for b in range(5):
                pltpu.async_copy(t_v.at[j + b], s_sh.at[dst_v.at[j + b]],
                                 tsem, add=True)
            for b in range(5):
                pltpu.make_async_copy(t_v.at[j + b],
                                      s_sh.at[dst_v.at[j + b]], tsem).wait()

    def _scale(rows, j):
        jsplat = jnp.full((16,), j, jnp.int32)

        @plsc.parallel_loop(0, CHUNK, step=4, unroll=4)
        def _(r):
            for rr in range(4):
                wv = plsc.load_gather(
                    t_v, [jsplat, jnp.full((16,), r + rr, jnp.int32)])
                for c in range(0, HC, 16):
                    rows[r + rr, pl.ds(c, 16)] = (
                        rows[r + rr, pl.ds(c, 16)] * wv)

    def _agg_loop(table_hbm):
        bufs = ((rows_a, gsem_a, ssem_a), (rows_b, gsem_b, ssem_b),
                (rows_c, gsem_c, ssem_c), (rows_d, gsem_d, ssem_d))
        nb = len(bufs)
        for b, (rows, gsem, _) in enumerate(bufs):
            pltpu.async_copy(table_hbm.at[src_v.at[b]], rows, gsem)

        @pl.loop(0, RPS // nb)
        def _(i):
            j = i * nb
            for b, (rows, gsem, ssem) in enumerate(bufs):
                jj = j + b
                pltpu.make_async_copy(table_hbm.at[src_v.at[jj]], rows,
                                      gsem).wait()
                _scale(rows, jj)
                pltpu.async_copy(rows, agg_sh.at[dst_v.at[jj]], ssem,
                                 add=True)

            @pl.when(i < RPS // nb - 1)
            def _():
                for b, (rows, gsem, ssem) in enumerate(bufs):
                    pltpu.make_async_copy(rows, agg_sh.at[dst_v.at[j + b]],
                                          ssem).wait()
                    pltpu.async_copy(table_hbm.at[src_v.at[j + nb + b]], rows,
                                     gsem)

            @pl.when(i == RPS // nb - 1)
            def _():
                for b, (rows, gsem, ssem) in enumerate(bufs):
                    pltpu.make_async_copy(rows, agg_sh.at[dst_v.at[j + b]],
                                          ssem).wait()

    @pl.when(cid == 0)
    def _():
        _agg_loop(srca_hbm)

    @pl.when(cid == 1)
    def _():
        _agg_loop(srcb_hbm)

    plsc.subcore_barrier()

    pltpu.sync_copy(agg_sh.at[pl.ds(sid * NPS, NPS)], agg_out.at[cid, sid])

    @pl.when(jnp.logical_and(cid == 0, sid == 0))
    def _():
        pltpu.sync_copy(s_sh, s_out.at[0])


def kernel(x, edge_index, batch, params):
    src3 = edge_index[0].reshape(NSUB, RPS, CHUNK)
    dst3 = edge_index[1].reshape(NSUB, RPS, CHUNK)
    zn = jnp.zeros((N,), _f32)

    def edge(al, ar, amax, srca, srcb):
        s_part, agg_part = _edge_kernel(src3, dst3, al.reshape(N),
                                        ar.reshape(N), amax,
                                        zn, srca, srcb)
        return s_part.reshape(N, 1), agg_part.reshape(NCORES, N, HC)

    srca, srcb, al, ar, amax = _prep0(x, params['Wl0'], params['bl0'],
                                      params['Wr0'], params['br0'],
                                      params['attl0'], params['attr0'])
    s_part, agg_part = edge(al, ar, amax, srca, srcb)
    for l in range(1, 5):
        srca, srcb, al, ar, amax = _prep(
            agg_part, s_part,
            params['Wl%d' % l], params['bl%d' % l],
            params['Wr%d' % l], params['br%d' % l],
            params['attl%d' % l], params['attr%d' % l])
        s_part, agg_part = edge(al, ar, amax, srca, srcb)
    return _final(agg_part, s_part,
                  params['Wp1'], params['bp1'], params['Wp2'], params['bp2'])


# speedup vs baseline: 1.1861x; 1.1861x over previous
import dataclasses
import functools

import jax
import jax.numpy as jnp
from jax import lax
from jax.experimental import pallas as pl
from jax.experimental.pallas import tpu as pltpu
from jax.experimental.pallas import tpu_sc as plsc

N = 10000
D = 128
E = 320000
NEG = 0.2
EPS = 1e-16

CHUNK = 80
EROWS = E // CHUNK
NCORES = 2
NSUB = 16
HC = D // NCORES
RPS = EROWS // NSUB
NCH = N // CHUNK
MPS = (NCH + NSUB - 1) // NSUB

_f32 = jnp.float32


def _prep_body(xa_ref, xb_ref, wl_ref, bl_ref, wr_ref, br_ref, attl_ref,
               attr_ref, srca_out, srcb_out, al_out, ar_out, amax_out):
    x = jnp.concatenate([xa_ref[...], xb_ref[...]], axis=1)
    source = jnp.dot(x, wl_ref[...], preferred_element_type=_f32)
    source = source + bl_ref[...][None, :]
    al = lax.dot_general(attl_ref[...], source, (((0,), (1,)), ((), ())),
                         preferred_element_type=_f32)
    wre = jnp.dot(wr_ref[...], attr_ref[...], preferred_element_type=_f32)
    bre = jnp.sum(br_ref[...] * attr_ref[...][:, 0])
    ar = lax.dot_general(wre, x, (((0,), (1,)), ((), ())),
                         preferred_element_type=_f32) + bre
    amax_out[...] = jnp.full((1, 16), jnp.max(al), _f32)
    srca_out[...] = source[:, :HC]
    srcb_out[...] = source[:, HC:]
    al_out[...] = al
    ar_out[...] = ar


def _final_body(xa_ref, xb_ref, wp1_ref, bp1_ref, wp2_ref, bp2_ref, out_ref):
    x = jnp.concatenate([xa_ref[...], xb_ref[...]], axis=1)
    h = jnp.dot(x, wp1_ref[...], preferred_element_type=_f32)
    h = h + bp1_ref[...][None, :]
    h = jnp.dot(h, wp2_ref[...], preferred_element_type=_f32)
    out_ref[...] = h + bp2_ref[...][None, :]


_prep_shapes = (
    jax.ShapeDtypeStruct((N, HC), _f32),
    jax.ShapeDtypeStruct((N, HC), _f32),
    jax.ShapeDtypeStruct((1, N), _f32),
    jax.ShapeDtypeStruct((1, N), _f32),
    jax.ShapeDtypeStruct((1, 16), _f32),
)

_prep = pl.pallas_call(_prep_body, out_shape=_prep_shapes)
_final = pl.pallas_call(_final_body,
                        out_shape=jax.ShapeDtypeStruct((N, D), _f32))


_mesh = plsc.VectorSubcoreMesh(core_axis_name="c", subcore_axis_name="s")

_sc_params = pltpu.CompilerParams()
for _k, _v in (("needs_layout_passes", False), ("use_tc_tiling_on_sc", False)):
    if _k in pltpu.CompilerParams.__dataclass_fields__:
        _sc_params = dataclasses.replace(_sc_params, **{_k: _v})


@functools.partial(
    pl.kernel,
    mesh=_mesh,
    compiler_params=_sc_params,
    out_type=jax.ShapeDtypeStruct((NCORES, NCH, CHUNK, HC), _f32),
    scratch_types=[
        pltpu.VMEM((RPS, CHUNK), jnp.int32),
        pltpu.VMEM((RPS, CHUNK), jnp.int32),
        pltpu.VMEM((RPS, CHUNK), _f32),
        pltpu.VMEM((N,), _f32),
        pltpu.VMEM((N,), _f32),
        pltpu.VMEM((1, 16), _f32),
        pltpu.VMEM((CHUNK, HC), _f32),
        pltpu.VMEM((CHUNK, HC), _f32),
        pltpu.VMEM((CHUNK,), _f32),
        pltpu.VMEM_SHARED((N,), _f32),
        pltpu.VMEM_SHARED((N, HC), _f32),
        pltpu.SemaphoreType.DMA,
        pltpu.SemaphoreType.DMA,
        pltpu.SemaphoreType.DMA,
        pltpu.SemaphoreType.DMA,
        pltpu.SemaphoreType.DMA,
    ],
)
def _edge_kernel(src_hbm, dst_hbm, al_hbm, ar_hbm, amax_hbm, zn_hbm,
                 srca_hbm, srcb_hbm, x_out,
                 src_v, dst_v, t_v, al_v, ar_v, amax_v, rows_a, rows_b, sv,
                 s_sh, agg_sh, gsem_a, gsem_b, ssem_a, ssem_b, tsem):
    cid = lax.axis_index("c")
    sid = lax.axis_index("s")

    _in_copies = ((src_hbm.at[sid], src_v), (dst_hbm.at[sid], dst_v),
                  (al_hbm.at[0], al_v), (ar_hbm.at[0], ar_v),
                  (amax_hbm, amax_v))
    for _src, _dst in _in_copies:
        pltpu.async_copy(_src, _dst, tsem)

    @pl.loop(0, CHUNK)
    def _(i):
        for c in range(0, HC, 16):
            rows_a[i, pl.ds(c, 16)] = jnp.zeros((16,), _f32)

    @pl.loop(0, MPS)
    def _(m):
        k = sid + m * NSUB

        @pl.when(k < NCH)
        def _():
            pltpu.async_copy(rows_a, agg_sh.at[pl.ds(k * CHUNK, CHUNK)],
                             gsem_b)

    @pl.when(sid == 0)
    def _():
        pltpu.sync_copy(zn_hbm, s_sh)

    for _src, _dst in _in_copies:
        pltpu.make_async_copy(_src, _dst, tsem).wait()

    av = amax_v[0, pl.ds(0, 16)]

    @plsc.parallel_loop(0, RPS, step=1, unroll=2)
    def _(j):
        for k in range(0, CHUNK, 16):
            si = src_v[j, pl.ds(k, 16)]
            di = dst_v[j, pl.ds(k, 16)]
            a = plsc.load_gather(al_v, [si])
            b = plsc.load_gather(ar_v, [di])
            z = a + b
            aw = jnp.where(z >= 0, z, NEG * z)
            z2 = b + av
            g = jnp.where(z2 >= 0, z2, NEG * z2)
            t_v[j, pl.ds(k, 16)] = jnp.exp(aw - g)

    @pl.loop(0, MPS)
    def _(m):
        k = sid + m * NSUB

        @pl.when(k < NCH)
        def _():
            pltpu.make_async_copy(rows_a,
                                  agg_sh.at[pl.ds(k * CHUNK, CHUNK)],
                                  gsem_b).wait()

    plsc.subcore_barrier()

    @pl.loop(0, RPS, step=10)
    def _(j):
        for b in range(10):
            pltpu.async_copy(t_v.at[j + b], s_sh.at[dst_v.at[j + b]],
                             tsem, add=True)
        for b in range(10):
            pltpu.make_async_copy(t_v.at[j + b],
                                  s_sh.at[dst_v.at[j + b]], tsem).wait()

    def _scale(rows, j):
        jsplat = jnp.full((16,), j, jnp.int32)

        @plsc.parallel_loop(0, CHUNK, step=4, unroll=4)
        def _(r):
            for rr in range(4):
                wv = plsc.load_gather(
                    t_v, [jsplat, jnp.full((16,), r + rr, jnp.int32)])
                for c in range(0, HC, 16):
                    rows[r + rr, pl.ds(c, 16)] = (
                        rows[r + rr, pl.ds(c, 16)] * wv)

    def _agg_loop(table_hbm):
        bufs = ((rows_a, gsem_a, ssem_a), (rows_b, gsem_b, ssem_b))
        for b, (rows, gsem, _) in enumerate(bufs):
            pltpu.async_copy(table_hbm.at[src_v.at[b]], rows, gsem)

        @pl.loop(0, RPS // 2)
        def _(i):
            j = i * 2
            for b, (rows, gsem, ssem) in enumerate(bufs):
                jj = j + b
                pltpu.make_async_copy(table_hbm.at[src_v.at[jj]], rows,
                                      gsem).wait()
                _scale(rows, jj)
                pltpu.async_copy(rows, agg_sh.at[dst_v.at[jj]], ssem,
                                 add=True)

            @pl.when(i < RPS // 2 - 1)
            def _():
                for b, (rows, gsem, ssem) in enumerate(bufs):
                    pltpu.make_async_copy(rows, agg_sh.at[dst_v.at[j + b]],
                                          ssem).wait()
                    pltpu.async_copy(table_hbm.at[src_v.at[j + 2 + b]], rows,
                                     gsem)

            @pl.when(i == RPS // 2 - 1)
            def _():
                for b, (rows, gsem, ssem) in enumerate(bufs):
                    pltpu.make_async_copy(rows, agg_sh.at[dst_v.at[j + b]],
                                          ssem).wait()

    @pl.when(cid == 0)
    def _():
        _agg_loop(srca_hbm)

    @pl.when(cid == 1)
    def _():
        _agg_loop(srcb_hbm)

    plsc.subcore_barrier()

    @pl.loop(0, MPS)
    def _(m):
        k = sid + m * NSUB

        @pl.when(k < NCH)
        def _():
            pltpu.sync_copy(agg_sh.at[pl.ds(k * CHUNK, CHUNK)], rows_a)
            pltpu.sync_copy(s_sh.at[pl.ds(k * CHUNK, CHUNK)], sv)

            for c in range(0, CHUNK, 16):
                sl = pl.ds(c, 16)
                sv[sl] = 1.0 / (sv[sl] + EPS)

            @plsc.parallel_loop(0, CHUNK, step=4, unroll=4)
            def _(r):
                for rr in range(4):
                    wv = plsc.load_gather(
                        sv, [jnp.full((16,), r + rr, jnp.int32)])
                    for c in range(0, HC, 16):
                        rows_a[r + rr, pl.ds(c, 16)] = jnp.maximum(
                            rows_a[r + rr, pl.ds(c, 16)] * wv, 0.0)

            pltpu.sync_copy(rows_a, x_out.at[cid, k])


def kernel(x, edge_index, batch, params):
    src3 = edge_index[0].reshape(NSUB, RPS, CHUNK)
    dst3 = edge_index[1].reshape(NSUB, RPS, CHUNK)
    zn = jnp.zeros((N,), _f32)

    ha, hb = x[:, :HC], x[:, HC:]
    for l in range(5):
        srca, srcb, al, ar, amax = _prep(ha, hb, params['Wl%d' % l],
                                         params['bl%d' % l],
                                         params['Wr%d' % l],
                                         params['br%d' % l],
                                         params['attl%d' % l],
                                         params['attr%d' % l])
        hh = _edge_kernel(src3, dst3, al, ar, amax, zn, srca, srcb)
        ha, hb = hh[0].reshape(N, HC), hh[1].reshape(N, HC)
    return _final(ha, hb, params['Wp1'], params['bp1'], params['Wp2'],
                  params['bp2'])
